# baseline (speedup 1.0000x reference)
import functools
import math

import jax
import jax.numpy as jnp
from jax import lax
from jax.experimental import pallas as pl
from jax.experimental.pallas import tpu as pltpu
from jax.experimental.pallas import tpu_sc as plsc

_SMOOTHING = 0.1
_CONFIDENCE = 1.0 - _SMOOTHING

_ROW_BLOCK = 128


def _tc_rowsum_body(x_ref, m_ref, out_ref):
    rs = jnp.sum(x_ref[...], axis=1)
    m = m_ref[0, 0, :]
    out_ref[0, 0, 0] = jnp.sum(rs * m)
    out_ref[0, 0, 1] = jnp.sum(m)


def _tc_masked_rowsum(x, m):
    n, v = x.shape
    grid = n // _ROW_BLOCK
    m3 = m.reshape(grid, 1, _ROW_BLOCK)
    return pl.pallas_call(
        _tc_rowsum_body,
        grid=(grid,),
        in_specs=[
            pl.BlockSpec((_ROW_BLOCK, v), lambda i: (i, 0)),
            pl.BlockSpec((1, 1, _ROW_BLOCK), lambda i: (i, 0, 0)),
        ],
        out_specs=pl.BlockSpec(
            (1, 1, 2), lambda i: (i, 0, 0), memory_space=pltpu.SMEM),
        out_shape=jax.ShapeDtypeStruct((grid, 1, 2), jnp.float32),
        compiler_params=pltpu.CompilerParams(
            dimension_semantics=("parallel",)),
    )(x, m3)


def _sc_masked_gather_partials(x_flat, flat_idx, m):
    n = flat_idx.shape[0]
    info = plsc.get_sparse_core_info()
    num_cores, num_subcores, num_lanes = (
        info.num_cores, info.num_subcores, info.num_lanes)
    nw = num_cores * num_subcores
    bpw = n // nw
    chunk = 128
    nchunks = bpw // chunk
    mesh = plsc.VectorSubcoreMesh(core_axis_name="c", subcore_axis_name="s")

    @functools.partial(
        pl.kernel,
        mesh=mesh,
        out_type=jax.ShapeDtypeStruct((nw, num_lanes), jnp.float32),
        scratch_types=[
            pltpu.VMEM((chunk,), jnp.int32),
            pltpu.VMEM((chunk,), jnp.float32),
            pltpu.VMEM((chunk,), jnp.float32),
            pltpu.VMEM((num_lanes,), jnp.float32),
            pltpu.SemaphoreType.DMA,
        ],
    )
    def k(x_hbm, idx_hbm, m_hbm, out_hbm, idx_v, vals_v, m_v, acc_v, sem):
        wid = lax.axis_index("s") * num_cores + lax.axis_index("c")
        base = wid * bpw
        acc = jnp.zeros((num_lanes,), jnp.float32)
        for c in range(nchunks):
            off = base + c * chunk
            pltpu.sync_copy(idx_hbm.at[pl.ds(off, chunk)], idx_v)
            pltpu.sync_copy(m_hbm.at[pl.ds(off, chunk)], m_v)
            pltpu.async_copy(x_hbm.at[idx_v], vals_v, sem).wait()
            for j in range(chunk // num_lanes):
                sl = pl.ds(j * num_lanes, num_lanes)
                acc = acc + vals_v[sl] * m_v[sl]
        acc_v[...] = acc
        pltpu.sync_copy(acc_v, out_hbm.at[wid])

    return k(x_flat, flat_idx, m)


def kernel(input, target, mask):
    b, t, v = input.shape
    n = b * t
    x = input.reshape(n, v)
    m = mask.reshape(n).astype(jnp.float32)
    tgt = target.reshape(n).astype(jnp.int32)

    xt = input.reshape(n // 8, 8, v // 128, 128)
    xt = xt.transpose(0, 2, 1, 3).reshape(-1)
    r = jnp.arange(n, dtype=jnp.int32)
    flat_idx = ((r // 8) * (8 * v) + (tgt // 128) * 1024
                + (r % 8) * 128 + (tgt % 128))

    tc_out = _tc_masked_rowsum(x, m)
    sc_partials = _sc_masked_gather_partials(xt, flat_idx, m)

    a = jnp.sum(tc_out[:, 0, 0])
    msum = jnp.sum(tc_out[:, 0, 1])
    tdot = jnp.sum(sc_partials)

    eps = _SMOOTHING / (v - 1)
    const = (v - 1) * eps * math.log(eps) + _CONFIDENCE * math.log(_CONFIDENCE)
    loss = (const * msum - eps * a - (_CONFIDENCE - eps) * tdot) / msum
    return loss


# speedup vs baseline: 1.0499x; 1.0499x over previous
import functools
import math

import jax
import jax.numpy as jnp
from jax import lax
from jax.experimental import pallas as pl
from jax.experimental.pallas import tpu as pltpu
from jax.experimental.pallas import tpu_sc as plsc

_SMOOTHING = 0.1
_CONFIDENCE = 1.0 - _SMOOTHING

_ROW_BLOCK = 256


def _tc_rowsum_body(x_ref, m_ref, out_ref):
    rs = jnp.sum(x_ref[...], axis=1)
    m = m_ref[0, 0, :]
    out_ref[0, 0, 0] = jnp.sum(rs * m)
    out_ref[0, 0, 1] = jnp.sum(m)


def _tc_masked_rowsum(x, m, grid):
    n, v = x.shape
    m3 = m.reshape(n // _ROW_BLOCK, 1, _ROW_BLOCK)
    return pl.pallas_call(
        _tc_rowsum_body,
        grid=(grid,),
        in_specs=[
            pl.BlockSpec((_ROW_BLOCK, v), lambda i: (i, 0)),
            pl.BlockSpec((1, 1, _ROW_BLOCK), lambda i: (i, 0, 0)),
        ],
        out_specs=pl.BlockSpec(
            (1, 1, 2), lambda i: (i, 0, 0), memory_space=pltpu.SMEM),
        out_shape=jax.ShapeDtypeStruct((grid, 1, 2), jnp.float32),
        compiler_params=pltpu.CompilerParams(
            dimension_semantics=("parallel",)),
    )(x, m3)


def _sc_masked_gather_partials(x_flat, flat_idx, m):
    n = flat_idx.shape[0]
    info = plsc.get_sparse_core_info()
    num_cores, num_subcores, num_lanes = (
        info.num_cores, info.num_subcores, info.num_lanes)
    nw = num_cores * num_subcores
    bpw = n // nw
    chunk = 128
    nchunks = bpw // chunk
    mesh = plsc.VectorSubcoreMesh(core_axis_name="c", subcore_axis_name="s")

    @functools.partial(
        pl.kernel,
        mesh=mesh,
        out_type=jax.ShapeDtypeStruct((nw, num_lanes), jnp.float32),
        scratch_types=[
            pltpu.VMEM((chunk,), jnp.int32),
            pltpu.VMEM((chunk,), jnp.float32),
            pltpu.VMEM((chunk,), jnp.float32),
            pltpu.VMEM((num_lanes,), jnp.float32),
            pltpu.SemaphoreType.DMA,
        ],
    )
    def k(x_hbm, idx_hbm, m_hbm, out_hbm, idx_v, vals_v, m_v, acc_v, sem):
        wid = lax.axis_index("s") * num_cores + lax.axis_index("c")
        base = wid * bpw
        acc = jnp.zeros((num_lanes,), jnp.float32)
        for c in range(nchunks):
            off = base + c * chunk
            pltpu.sync_copy(idx_hbm.at[pl.ds(off, chunk)], idx_v)
            pltpu.sync_copy(m_hbm.at[pl.ds(off, chunk)], m_v)
            pltpu.async_copy(x_hbm.at[idx_v], vals_v, sem).wait()
            for j in range(chunk // num_lanes):
                sl = pl.ds(j * num_lanes, num_lanes)
                acc = acc + vals_v[sl] * m_v[sl]
        acc_v[...] = acc
        pltpu.sync_copy(acc_v, out_hbm.at[wid])

    return k(x_flat, flat_idx, m)


def _sc_dense_partials(xt, mexp, band0, nb):
    info = plsc.get_sparse_core_info()
    num_cores, num_subcores, num_lanes = (
        info.num_cores, info.num_subcores, info.num_lanes)
    nw = num_cores * num_subcores
    chunk = 32768
    nchunks = nb * 2
    mesh = plsc.VectorSubcoreMesh(core_axis_name="c", subcore_axis_name="s")

    @functools.partial(
        pl.kernel,
        mesh=mesh,
        out_type=jax.ShapeDtypeStruct((nw, num_lanes), jnp.float32),
        scratch_types=[
            pltpu.VMEM((2, chunk), jnp.float32),
            pltpu.VMEM((nb * 8 * num_lanes,), jnp.float32),
            pltpu.VMEM((num_lanes,), jnp.float32),
            pltpu.SemaphoreType.DMA,
            pltpu.SemaphoreType.DMA,
        ],
    )
    def k(x_hbm, m_hbm, out_hbm, vals, mband, stage, sem0, sem1):
        wid = lax.axis_index("s") * num_cores + lax.axis_index("c")
        my_band0 = band0 + wid * nb
        mrow = nb * 8 * num_lanes
        pltpu.sync_copy(m_hbm.at[pl.ds(wid * mrow, mrow)], mband)
        bufs = (vals.at[0], vals.at[1])
        sems = (sem0, sem1)

        def start(g):
            j, h = divmod(g, 2)
            off = (my_band0 + j) * 65536 + h * chunk
            return pltpu.async_copy(
                x_hbm.at[pl.ds(off, chunk)], bufs[g % 2], sems[g % 2])

        pending = {0: start(0)}
        vtot = jnp.zeros((num_lanes,), jnp.float32)
        accs = None
        for g in range(nchunks):
            if g + 1 < nchunks:
                pending[g + 1] = start(g + 1)
            pending.pop(g).wait()
            j, h = divmod(g, 2)
            if h == 0:
                accs = tuple(jnp.zeros((num_lanes,), jnp.float32)
                             for _ in range(8))
            buf = bufs[g % 2]

            def tile_body(t, a, buf=buf):
                new = []
                for c in range(8):
                    acc = a[c]
                    for kk in range(8):
                        acc = acc + buf[pl.ds(t * 1024 + c * 128 + kk * 16,
                                              num_lanes)]
                    new.append(acc)
                return tuple(new)

            accs = lax.fori_loop(0, 32, tile_body, accs)
            if h == 1:
                for c in range(8):
                    mc = mband[pl.ds((j * 8 + c) * num_lanes, num_lanes)]
                    vtot = vtot + accs[c] * mc
        stage[...] = vtot
        pltpu.sync_copy(stage, out_hbm.at[wid])

    return k(xt, mexp)


_SC_BANDS_PER_WORKER = 10


def kernel(input, target, mask):
    b, t, v = input.shape
    n = b * t
    x = input.reshape(n, v)
    m = mask.reshape(n).astype(jnp.float32)
    tgt = target.reshape(n).astype(jnp.int32)

    xt = input.reshape(n // 8, 8, v // 128, 128)
    xt = xt.transpose(0, 2, 1, 3).reshape(-1)
    r = jnp.arange(n, dtype=jnp.int32)
    flat_idx = ((r // 8) * (8 * v) + (tgt // 128) * 1024
                + (r % 8) * 128 + (tgt % 128))

    nb = _SC_BANDS_PER_WORKER
    n_bands = n // 8
    band0 = n_bands - 32 * nb
    tc_grid = band0 * 8 // _ROW_BLOCK

    mexp = jnp.repeat(m[band0 * 8:], 16)
    tc_out = _tc_masked_rowsum(x, m, tc_grid)
    sc_partials = _sc_masked_gather_partials(xt, flat_idx, m)
    sc_dense = _sc_dense_partials(xt, mexp, band0, nb)

    a = jnp.sum(tc_out[:, 0, 0]) + jnp.sum(sc_dense)
    msum = jnp.sum(tc_out[:, 0, 1]) + jnp.sum(m[band0 * 8:])
    tdot = jnp.sum(sc_partials)

    eps = _SMOOTHING / (v - 1)
    const = (v - 1) * eps * math.log(eps) + _CONFIDENCE * math.log(_CONFIDENCE)
    loss = (const * msum - eps * a - (_CONFIDENCE - eps) * tdot) / msum
    return loss


# speedup vs baseline: 1.0508x; 1.0008x over previous
import functools
import math

import jax
import jax.numpy as jnp
from jax import lax
from jax.experimental import pallas as pl
from jax.experimental.pallas import tpu as pltpu
from jax.experimental.pallas import tpu_sc as plsc

_SMOOTHING = 0.1
_CONFIDENCE = 1.0 - _SMOOTHING

_ROW_BLOCK = 256


def _tc_rowsum_body(x_ref, m_ref, out_ref):
    rs = jnp.sum(x_ref[...], axis=1)
    m = m_ref[0, 0, :]
    out_ref[0, 0, 0] = jnp.sum(rs * m)
    out_ref[0, 0, 1] = jnp.sum(m)


def _tc_masked_rowsum(x, m, grid):
    n, v = x.shape
    m3 = m.reshape(n // _ROW_BLOCK, 1, _ROW_BLOCK)
    return pl.pallas_call(
        _tc_rowsum_body,
        grid=(grid,),
        in_specs=[
            pl.BlockSpec((_ROW_BLOCK, v), lambda i: (i, 0)),
            pl.BlockSpec((1, 1, _ROW_BLOCK), lambda i: (i, 0, 0)),
        ],
        out_specs=pl.BlockSpec(
            (1, 1, 2), lambda i: (i, 0, 0), memory_space=pltpu.SMEM),
        out_shape=jax.ShapeDtypeStruct((grid, 1, 2), jnp.float32),
        compiler_params=pltpu.CompilerParams(
            dimension_semantics=("parallel",)),
    )(x, m3)


def _sc_masked_gather_partials(x_flat, flat_idx, m):
    n = flat_idx.shape[0]
    info = plsc.get_sparse_core_info()
    num_cores, num_subcores, num_lanes = (
        info.num_cores, info.num_subcores, info.num_lanes)
    nw = num_cores * num_subcores
    bpw = n // nw
    chunk = 128
    nchunks = bpw // chunk
    mesh = plsc.VectorSubcoreMesh(core_axis_name="c", subcore_axis_name="s")

    @functools.partial(
        pl.kernel,
        mesh=mesh,
        out_type=jax.ShapeDtypeStruct((nw, num_lanes), jnp.float32),
        scratch_types=[
            pltpu.VMEM((chunk,), jnp.int32),
            pltpu.VMEM((chunk,), jnp.float32),
            pltpu.VMEM((chunk,), jnp.float32),
            pltpu.VMEM((num_lanes,), jnp.float32),
            pltpu.SemaphoreType.DMA,
        ],
    )
    def k(x_hbm, idx_hbm, m_hbm, out_hbm, idx_v, vals_v, m_v, acc_v, sem):
        wid = lax.axis_index("s") * num_cores + lax.axis_index("c")
        base = wid * bpw
        acc = jnp.zeros((num_lanes,), jnp.float32)
        for c in range(nchunks):
            off = base + c * chunk
            pltpu.sync_copy(idx_hbm.at[pl.ds(off, chunk)], idx_v)
            pltpu.sync_copy(m_hbm.at[pl.ds(off, chunk)], m_v)
            pltpu.async_copy(x_hbm.at[idx_v], vals_v, sem).wait()
            for j in range(chunk // num_lanes):
                sl = pl.ds(j * num_lanes, num_lanes)
                acc = acc + vals_v[sl] * m_v[sl]
        acc_v[...] = acc
        pltpu.sync_copy(acc_v, out_hbm.at[wid])

    return k(x_flat, flat_idx, m)


def _sc_dense_partials(xt, mexp, band0, nb):
    info = plsc.get_sparse_core_info()
    num_cores, num_subcores, num_lanes = (
        info.num_cores, info.num_subcores, info.num_lanes)
    nw = num_cores * num_subcores
    chunk = 32768
    nchunks = nb * 2
    mesh = plsc.VectorSubcoreMesh(core_axis_name="c", subcore_axis_name="s")

    @functools.partial(
        pl.kernel,
        mesh=mesh,
        out_type=jax.ShapeDtypeStruct((nw, num_lanes), jnp.float32),
        scratch_types=[
            pltpu.VMEM((2, chunk), jnp.float32),
            pltpu.VMEM((nb * 8 * num_lanes,), jnp.float32),
            pltpu.VMEM((num_lanes,), jnp.float32),
            pltpu.SemaphoreType.DMA,
            pltpu.SemaphoreType.DMA,
        ],
    )
    def k(x_hbm, m_hbm, out_hbm, vals, mband, stage, sem0, sem1):
        wid = lax.axis_index("s") * num_cores + lax.axis_index("c")
        my_band0 = band0 + wid * nb
        mrow = nb * 8 * num_lanes
        pltpu.sync_copy(m_hbm.at[pl.ds(wid * mrow, mrow)], mband)
        bufs = (vals.at[0], vals.at[1])
        sems = (sem0, sem1)

        def start(g):
            j, h = divmod(g, 2)
            off = (my_band0 + j) * 65536 + h * chunk
            return pltpu.async_copy(
                x_hbm.at[pl.ds(off, chunk)], bufs[g % 2], sems[g % 2])

        pending = {0: start(0)}
        vtot = jnp.zeros((num_lanes,), jnp.float32)
        accs = None
        for g in range(nchunks):
            if g + 1 < nchunks:
                pending[g + 1] = start(g + 1)
            pending.pop(g).wait()
            j, h = divmod(g, 2)
            if h == 0:
                accs = tuple(jnp.zeros((num_lanes,), jnp.float32)
                             for _ in range(8))
            buf = bufs[g % 2]

            def tile_body(t, a, buf=buf):
                new = list(a)
                for kk in range(8):
                    for c in range(8):
                        new[c] = new[c] + buf[pl.ds(
                            t * 1024 + c * 128 + kk * 16, num_lanes)]
                return tuple(new)

            accs = lax.fori_loop(0, 32, tile_body, accs)
            if h == 1:
                for c in range(8):
                    mc = mband[pl.ds((j * 8 + c) * num_lanes, num_lanes)]
                    vtot = vtot + accs[c] * mc
        stage[...] = vtot
        pltpu.sync_copy(stage, out_hbm.at[wid])

    return k(xt, mexp)


_SC_BANDS_PER_WORKER = 10


def kernel(input, target, mask):
    b, t, v = input.shape
    n = b * t
    x = input.reshape(n, v)
    m = mask.reshape(n).astype(jnp.float32)
    tgt = target.reshape(n).astype(jnp.int32)

    xt = input.reshape(n // 8, 8, v // 128, 128)
    xt = xt.transpose(0, 2, 1, 3).reshape(-1)
    r = jnp.arange(n, dtype=jnp.int32)
    flat_idx = ((r // 8) * (8 * v) + (tgt // 128) * 1024
                + (r % 8) * 128 + (tgt % 128))

    nb = _SC_BANDS_PER_WORKER
    n_bands = n // 8
    band0 = n_bands - 32 * nb
    tc_grid = band0 * 8 // _ROW_BLOCK

    mexp = jnp.repeat(m[band0 * 8:], 16)
    tc_out = _tc_masked_rowsum(x, m, tc_grid)
    sc_partials = _sc_masked_gather_partials(xt, flat_idx, m)
    sc_dense = _sc_dense_partials(xt, mexp, band0, nb)

    a = jnp.sum(tc_out[:, 0, 0]) + jnp.sum(sc_dense)
    msum = jnp.sum(tc_out[:, 0, 1]) + jnp.sum(m[band0 * 8:])
    tdot = jnp.sum(sc_partials)

    eps = _SMOOTHING / (v - 1)
    const = (v - 1) * eps * math.log(eps) + _CONFIDENCE * math.log(_CONFIDENCE)
    loss = (const * msum - eps * a - (_CONFIDENCE - eps) * tdot) / msum
    return loss


# speedup vs baseline: 1.1413x; 1.0862x over previous
import functools
import math

import jax
import jax.numpy as jnp
from jax import lax
from jax.experimental import pallas as pl
from jax.experimental.pallas import tpu as pltpu
from jax.experimental.pallas import tpu_sc as plsc

_SMOOTHING = 0.1
_CONFIDENCE = 1.0 - _SMOOTHING

_ROW_BLOCK = 256


def _tc_rowsum_body(x_ref, m_ref, out_ref):
    rs = jnp.sum(x_ref[...], axis=1)
    m = m_ref[...]
    out_ref[0, 0, 0] = jnp.sum(rs * m)
    out_ref[0, 0, 1] = jnp.sum(m)


def _tc_masked_rowsum(x, m):
    n, v = x.shape
    grid = n // _ROW_BLOCK
    return pl.pallas_call(
        _tc_rowsum_body,
        grid=(grid,),
        in_specs=[
            pl.BlockSpec((_ROW_BLOCK, v), lambda i: (i, 0)),
            pl.BlockSpec((_ROW_BLOCK,), lambda i: (i,)),
        ],
        out_specs=pl.BlockSpec(
            (1, 1, 2), lambda i: (i, 0, 0), memory_space=pltpu.SMEM),
        out_shape=jax.ShapeDtypeStruct((grid, 1, 2), jnp.float32),
        compiler_params=pltpu.CompilerParams(
            dimension_semantics=("parallel",)),
    )(x, m)


def _sc_masked_gather_partials(x_flat, flat_idx, m):
    n = flat_idx.shape[0]
    info = plsc.get_sparse_core_info()
    num_cores, num_subcores, num_lanes = (
        info.num_cores, info.num_subcores, info.num_lanes)
    nw = num_cores * num_subcores
    bpw = n // nw
    chunk = 128
    nchunks = bpw // chunk
    mesh = plsc.VectorSubcoreMesh(core_axis_name="c", subcore_axis_name="s")

    @functools.partial(
        pl.kernel,
        mesh=mesh,
        out_type=jax.ShapeDtypeStruct((nw, num_lanes), jnp.float32),
        scratch_types=[
            pltpu.VMEM((chunk,), jnp.int32),
            pltpu.VMEM((chunk,), jnp.float32),
            pltpu.VMEM((chunk,), jnp.float32),
            pltpu.VMEM((num_lanes,), jnp.float32),
            pltpu.SemaphoreType.DMA,
        ],
    )
    def k(x_hbm, idx_hbm, m_hbm, out_hbm, idx_v, vals_v, m_v, acc_v, sem):
        wid = lax.axis_index("s") * num_cores + lax.axis_index("c")
        base = wid * bpw
        acc = jnp.zeros((num_lanes,), jnp.float32)
        for c in range(nchunks):
            off = base + c * chunk
            pltpu.sync_copy(idx_hbm.at[pl.ds(off, chunk)], idx_v)
            pltpu.sync_copy(m_hbm.at[pl.ds(off, chunk)], m_v)
            pltpu.async_copy(x_hbm.at[idx_v], vals_v, sem).wait()
            for j in range(chunk // num_lanes):
                sl = pl.ds(j * num_lanes, num_lanes)
                acc = acc + vals_v[sl] * m_v[sl]
        acc_v[...] = acc
        pltpu.sync_copy(acc_v, out_hbm.at[wid])

    return k(x_flat, flat_idx, m)


def kernel(input, target, mask):
    b, t, v = input.shape
    n = b * t
    x = input.reshape(n, v)
    m = mask.reshape(n).astype(jnp.float32)
    tgt = target.reshape(n).astype(jnp.int32)

    xt = input.reshape(n // 8, 8, v // 128, 128)
    xt = xt.transpose(0, 2, 1, 3).reshape(-1)
    r = jnp.arange(n, dtype=jnp.int32)
    flat_idx = ((r // 8) * (8 * v) + (tgt // 128) * 1024
                + (r % 8) * 128 + (tgt % 128))

    tc_out = _tc_masked_rowsum(x, m)
    sc_partials = _sc_masked_gather_partials(xt, flat_idx, m)

    a = jnp.sum(tc_out[:, 0, 0])
    msum = jnp.sum(tc_out[:, 0, 1])
    tdot = jnp.sum(sc_partials)

    eps = _SMOOTHING / (v - 1)
    const = (v - 1) * eps * math.log(eps) + _CONFIDENCE * math.log(_CONFIDENCE)
    loss = (const * msum - eps * a - (_CONFIDENCE - eps) * tdot) / msum
    return loss
